# Initial kernel scaffold; baseline (speedup 1.0000x reference)
import jax
import jax.numpy as jnp
from jax.experimental import pallas as pl


def kernel(x_user, x_item, edge_index_rates, edge_index_rev, Wp_user, bp_user, Wp_item, bp_item, Wl0_rates, bl0_rates, Wr0_rates, Wl0_rev, bl0_rev, Wr0_rev, Wl1_rates, bl1_rates, Wr1_rates, Wl1_rev, bl1_rev, Wr1_rev, Wf_user, bf_user, Wf_item, bf_item):
    raise NotImplementedError("write your pallas kernel here")


# speedup vs baseline: 8.0415x; 8.0415x over previous
import functools

import jax
import jax.numpy as jnp
from jax import lax
from jax.experimental import pallas as pl
from jax.experimental.pallas import tpu as pltpu
from jax.experimental.pallas import tpu_sc as plsc

NU = 10000
NI = 10000
DIN = 128
H = 64
N = 10000
NC = 2
NS = 16
L = 16
E = 320000
CH = 128
NCHUNK = -(-E // (NS * CH))
EPT = NCHUNK * CH
EPAD = NS * EPT
RPT = 640
NACC = NS * RPT
DPAD = N + 8
RB = 1000


@functools.cache
def _make_segsum(with_counts: bool):
    mesh = plsc.VectorSubcoreMesh(
        core_axis_name="c", subcore_axis_name="s", num_cores=NC,
        num_subcores=NS)

    out_type = [jax.ShapeDtypeStruct((NC, NACC, H), jnp.float32)]
    scratch = [
        pltpu.VMEM((NCHUNK, CH), jnp.int32),
        pltpu.VMEM((NCHUNK, CH), jnp.int32),
        pltpu.VMEM((CH, H), jnp.float32),
        pltpu.VMEM((64, H), jnp.float32),
        pltpu.VMEM_SHARED((NACC, H), jnp.float32),
        pltpu.SemaphoreType.DMA,
    ]
    if with_counts:
        out_type.append(jax.ShapeDtypeStruct((NC, NS, RPT, L), jnp.float32))
        scratch += [
            pltpu.VMEM((CH, L), jnp.float32),
            pltpu.VMEM((RPT, L), jnp.float32),
            pltpu.VMEM_SHARED((NACC, L), jnp.float32),
        ]

    def body(tab, idx_src, idx_dst, *rest):
        if with_counts:
            (out_s, out_inv, src_v, dst_v, rows_v, zb_v, acc_sh, sem,
             ones_v, inv_v, cnt_sh) = rest
        else:
            out_s, src_v, dst_v, rows_v, zb_v, acc_sh, sem = rest

        c = lax.axis_index("c")
        s = lax.axis_index("s")
        w = c * NS + s
        zeros16 = jnp.zeros((L,), jnp.float32)

        def zb_loop(i, carry):
            for k in range(H // L):
                zb_v[i, pl.ds(k * L, L)] = zeros16
            return carry
        lax.fori_loop(0, 64, zb_loop, 0)
        for t in range(RPT // 64):
            pltpu.sync_copy(zb_v, acc_sh.at[pl.ds(s * RPT + t * 64, 64)])

        if with_counts:
            onehot = jnp.where(lax.iota(jnp.int32, L) == 0,
                               jnp.float32(1), jnp.float32(0))

            def oh_loop(i, carry):
                ones_v[i, pl.ds(0, L)] = onehot
                return carry
            lax.fori_loop(0, CH, oh_loop, 0)

            def zc_loop(i, carry):
                inv_v[i, pl.ds(0, L)] = zeros16
                return carry
            lax.fori_loop(0, RPT, zc_loop, 0)
            pltpu.sync_copy(inv_v, cnt_sh.at[pl.ds(s * RPT, RPT)])

        pltpu.sync_copy(idx_src.at[w], src_v)
        pltpu.sync_copy(idx_dst.at[w], dst_v)
        plsc.subcore_barrier()

        def edge_loop(j, carry):
            pltpu.async_copy(tab.at[src_v.at[j]], rows_v, sem).wait()
            pltpu.sync_copy(rows_v, acc_sh.at[dst_v.at[j]], add=True)
            if with_counts:
                pltpu.sync_copy(ones_v, cnt_sh.at[dst_v.at[j]], add=True)
            return carry
        lax.fori_loop(0, NCHUNK, edge_loop, 0)

        plsc.subcore_barrier()

        pltpu.sync_copy(acc_sh.at[pl.ds(s * RPT, RPT)],
                        out_s.at[c, pl.ds(s * RPT, RPT)])
        if with_counts:
            pltpu.sync_copy(cnt_sh.at[pl.ds(s * RPT, RPT)], inv_v)

            def inv_loop(i, carry):
                v = inv_v[i, pl.ds(0, L)]
                inv_v[i, pl.ds(0, L)] = 1.0 / jnp.maximum(v, 1.0)
                return carry
            lax.fori_loop(0, RPT, inv_loop, 0)
            pltpu.sync_copy(inv_v, out_inv.at[c, s])

    return pl.kernel(
        body, out_type=out_type, mesh=mesh, scratch_types=scratch,
        compiler_params=pltpu.CompilerParams(use_tc_tiling_on_sc=False))


def _full(shape):
    return pl.BlockSpec(shape, lambda i: tuple(0 for _ in shape))


def _tc1_body(xu, xi, wpu, bpu, wpi, bpi, wl_r, wr_r, blr, wl_v, wr_v, blv,
              tab, root):
    hu = jax.nn.relu(jnp.dot(xu[...], wpu[...],
                             preferred_element_type=jnp.float32) + bpu[...])
    hi = jax.nn.relu(jnp.dot(xi[...], wpi[...],
                             preferred_element_type=jnp.float32) + bpi[...])
    tab[0] = jnp.dot(hu, wl_r[...], preferred_element_type=jnp.float32)
    tab[1] = jnp.dot(hi, wl_v[...], preferred_element_type=jnp.float32)
    root[0] = jnp.dot(hi, wr_r[...],
                      preferred_element_type=jnp.float32) + blr[...]
    root[1] = jnp.dot(hu, wr_v[...],
                      preferred_element_type=jnp.float32) + blv[...]


def _tc2_body(ssum, inv, root0, wl_r, wr_r, blr, wl_v, wr_v, blv, tab, root):
    hi = jax.nn.relu(ssum[0] * inv[0] + root0[0])
    hu = jax.nn.relu(ssum[1] * inv[1] + root0[1])
    tab[0] = jnp.dot(hu, wl_r[...], preferred_element_type=jnp.float32)
    tab[1] = jnp.dot(hi, wl_v[...], preferred_element_type=jnp.float32)
    root[0] = jnp.dot(hi, wr_r[...],
                      preferred_element_type=jnp.float32) + blr[...]
    root[1] = jnp.dot(hu, wr_v[...],
                      preferred_element_type=jnp.float32) + blv[...]


def _tc3_body(ssum, inv, root1, wfu, bfu, wfi, bfi, out_u, out_i):
    hi = jax.nn.relu(ssum[0] * inv[0] + root1[0])
    hu = jax.nn.relu(ssum[1] * inv[1] + root1[1])
    out_u[...] = jnp.dot(hu, wfu[...],
                         preferred_element_type=jnp.float32) + bfu[...]
    out_i[...] = jnp.dot(hi, wfi[...],
                         preferred_element_type=jnp.float32) + bfi[...]


_G = N // RB
_b2 = pl.BlockSpec((2, RB, H), lambda i: (0, i, 0))
_b2i = pl.BlockSpec((2, RB, 1), lambda i: (0, i, 0))
_b1 = pl.BlockSpec((RB, H), lambda i: (i, 0))

_tc1 = pl.pallas_call(
    _tc1_body,
    grid=(_G,),
    in_specs=[
        pl.BlockSpec((RB, DIN), lambda i: (i, 0)),
        pl.BlockSpec((RB, DIN), lambda i: (i, 0)),
        _full((DIN, H)), _full((1, H)), _full((DIN, H)), _full((1, H)),
        _full((H, H)), _full((H, H)), _full((1, H)),
        _full((H, H)), _full((H, H)), _full((1, H)),
    ],
    out_specs=[_b2, _b2],
    out_shape=[jax.ShapeDtypeStruct((2, N, H), jnp.float32)] * 2,
)

_tc2 = pl.pallas_call(
    _tc2_body,
    grid=(_G,),
    in_specs=[
        _b2, _b2i, _b2,
        _full((H, H)), _full((H, H)), _full((1, H)),
        _full((H, H)), _full((H, H)), _full((1, H)),
    ],
    out_specs=[_b2, _b2],
    out_shape=[jax.ShapeDtypeStruct((2, N, H), jnp.float32)] * 2,
)

_tc3 = pl.pallas_call(
    _tc3_body,
    grid=(_G,),
    in_specs=[
        _b2, _b2i, _b2,
        _full((H, H)), _full((1, H)), _full((H, H)), _full((1, H)),
    ],
    out_specs=[_b1, _b1],
    out_shape=[jax.ShapeDtypeStruct((N, H), jnp.float32)] * 2,
)


def _pad_edges(src, dst, src_off):
    pad = EPAD - E
    src = jnp.concatenate(
        [src.astype(jnp.int32) + src_off, jnp.zeros((pad,), jnp.int32)])
    dst = jnp.concatenate(
        [dst.astype(jnp.int32), jnp.full((pad,), DPAD, jnp.int32)])
    return src.reshape(NS, NCHUNK, CH), dst.reshape(NS, NCHUNK, CH)


@jax.jit
def kernel(x_user, x_item, edge_index_rates, edge_index_rev,
           Wp_user, bp_user, Wp_item, bp_item,
           Wl0_rates, bl0_rates, Wr0_rates, Wl0_rev, bl0_rev, Wr0_rev,
           Wl1_rates, bl1_rates, Wr1_rates, Wl1_rev, bl1_rev, Wr1_rev,
           Wf_user, bf_user, Wf_item, bf_item):
    src_r, dst_r = _pad_edges(edge_index_rates[0], edge_index_rates[1], 0)
    src_v, dst_v = _pad_edges(edge_index_rev[0], edge_index_rev[1], N)
    idx_src = jnp.concatenate([src_r[None], src_v[None]]).reshape(
        NC * NS, NCHUNK, CH)
    idx_dst = jnp.concatenate([dst_r[None], dst_v[None]]).reshape(
        NC * NS, NCHUNK, CH)

    b2 = lambda b: b.reshape(1, H)
    tab0, root0 = _tc1(x_user, x_item, Wp_user, b2(bp_user), Wp_item,
                       b2(bp_item), Wl0_rates, Wr0_rates, b2(bl0_rates),
                       Wl0_rev, Wr0_rev, b2(bl0_rev))

    s0, inv_raw = _make_segsum(True)(tab0.reshape(NC * N, H), idx_src,
                                     idx_dst)
    inv = inv_raw.reshape(NC, NACC, L)[:, :, :1]

    tab1, root1 = _tc2(s0, inv, root0, Wl1_rates, Wr1_rates, b2(bl1_rates),
                       Wl1_rev, Wr1_rev, b2(bl1_rev))

    (s1,) = _make_segsum(False)(tab1.reshape(NC * N, H), idx_src, idx_dst)

    out_user, out_item = _tc3(s1, inv, root1, Wf_user, b2(bf_user),
                              Wf_item, b2(bf_item))
    return (out_user, out_item)


# speedup vs baseline: 10.8317x; 1.3470x over previous
import functools

import jax
import jax.numpy as jnp
from jax import lax
from jax.experimental import pallas as pl
from jax.experimental.pallas import tpu as pltpu
from jax.experimental.pallas import tpu_sc as plsc

NU = 10000
NI = 10000
DIN = 128
H = 64
N = 10000
NC = 2
NS = 16
L = 16
E = 320000
CH = 128
NCHUNK = -(-E // (NS * CH))
EPT = NCHUNK * CH
EPAD = NS * EPT
RPT = 640
NACC = NS * RPT
DPAD = N + 8
RB = 1000


@functools.cache
def _make_segsum(with_counts: bool):
    mesh = plsc.VectorSubcoreMesh(
        core_axis_name="c", subcore_axis_name="s", num_cores=NC,
        num_subcores=NS)

    out_type = [jax.ShapeDtypeStruct((NC, NACC, H), jnp.float32)]
    scratch = [
        pltpu.VMEM((NCHUNK, CH), jnp.int32),
        pltpu.VMEM((NCHUNK, CH), jnp.int32),
        pltpu.VMEM((2, CH, H), jnp.float32),
        pltpu.VMEM((64, H), jnp.float32),
        pltpu.VMEM_SHARED((NACC, H), jnp.float32),
        pltpu.SemaphoreType.DMA((2,)),
    ]
    if with_counts:
        out_type.append(jax.ShapeDtypeStruct((NC, NS, RPT, L), jnp.float32))
        scratch += [
            pltpu.VMEM((CH, L), jnp.float32),
            pltpu.VMEM((RPT, L), jnp.float32),
            pltpu.VMEM_SHARED((NACC, L), jnp.float32),
        ]

    def body(tab, idx_src, idx_dst, *rest):
        if with_counts:
            (out_s, out_inv, src_v, dst_v, rows_v, zb_v, acc_sh, sem,
             ones_v, inv_v, cnt_sh) = rest
        else:
            out_s, src_v, dst_v, rows_v, zb_v, acc_sh, sem = rest

        c = lax.axis_index("c")
        s = lax.axis_index("s")
        w = c * NS + s
        zeros16 = jnp.zeros((L,), jnp.float32)

        def zb_loop(i, carry):
            for k in range(H // L):
                zb_v[i, pl.ds(k * L, L)] = zeros16
            return carry
        lax.fori_loop(0, 64, zb_loop, 0)
        for t in range(RPT // 64):
            pltpu.sync_copy(zb_v, acc_sh.at[pl.ds(s * RPT + t * 64, 64)])

        if with_counts:
            onehot = jnp.where(lax.iota(jnp.int32, L) == 0,
                               jnp.float32(1), jnp.float32(0))

            def oh_loop(i, carry):
                ones_v[i, pl.ds(0, L)] = onehot
                return carry
            lax.fori_loop(0, CH, oh_loop, 0)

            def zc_loop(i, carry):
                inv_v[i, pl.ds(0, L)] = zeros16
                return carry
            lax.fori_loop(0, RPT, zc_loop, 0)
            pltpu.sync_copy(inv_v, cnt_sh.at[pl.ds(s * RPT, RPT)])

        pltpu.sync_copy(idx_src.at[w], src_v)
        pltpu.sync_copy(idx_dst.at[w], dst_v)
        plsc.subcore_barrier()

        pltpu.async_copy(tab.at[src_v.at[0]], rows_v.at[0], sem.at[0])

        def edge_loop(j, carry):
            b = lax.rem(j, 2)
            nb = lax.rem(j + 1, 2)

            @pl.when(j + 1 < NCHUNK)
            def _():
                pltpu.async_copy(tab.at[src_v.at[j + 1]], rows_v.at[nb],
                                 sem.at[nb])
            pltpu.make_async_copy(tab.at[src_v.at[j]], rows_v.at[b],
                                  sem.at[b]).wait()
            pltpu.sync_copy(rows_v.at[b], acc_sh.at[dst_v.at[j]], add=True)
            if with_counts:
                pltpu.sync_copy(ones_v, cnt_sh.at[dst_v.at[j]], add=True)
            return carry
        lax.fori_loop(0, NCHUNK, edge_loop, 0)

        plsc.subcore_barrier()

        pltpu.sync_copy(acc_sh.at[pl.ds(s * RPT, RPT)],
                        out_s.at[c, pl.ds(s * RPT, RPT)])
        if with_counts:
            pltpu.sync_copy(cnt_sh.at[pl.ds(s * RPT, RPT)], inv_v)

            def inv_loop(i, carry):
                v = inv_v[i, pl.ds(0, L)]
                inv_v[i, pl.ds(0, L)] = 1.0 / jnp.maximum(v, 1.0)
                return carry
            lax.fori_loop(0, RPT, inv_loop, 0)
            pltpu.sync_copy(inv_v, out_inv.at[c, s])

    return pl.kernel(
        body, out_type=out_type, mesh=mesh, scratch_types=scratch,
        compiler_params=pltpu.CompilerParams(use_tc_tiling_on_sc=False))


def _full(shape):
    return pl.BlockSpec(shape, lambda i: tuple(0 for _ in shape))


def _tc1_body(xu, xi, wpu, bpu, wpi, bpi, wl_r, wr_r, blr, wl_v, wr_v, blv,
              tab, root):
    hu = jax.nn.relu(jnp.dot(xu[...], wpu[...],
                             preferred_element_type=jnp.float32) + bpu[...])
    hi = jax.nn.relu(jnp.dot(xi[...], wpi[...],
                             preferred_element_type=jnp.float32) + bpi[...])
    tab[0] = jnp.dot(hu, wl_r[...], preferred_element_type=jnp.float32)
    tab[1] = jnp.dot(hi, wl_v[...], preferred_element_type=jnp.float32)
    root[0] = jnp.dot(hi, wr_r[...],
                      preferred_element_type=jnp.float32) + blr[...]
    root[1] = jnp.dot(hu, wr_v[...],
                      preferred_element_type=jnp.float32) + blv[...]


def _tc2_body(ssum, inv, root0, wl_r, wr_r, blr, wl_v, wr_v, blv, tab, root):
    hi = jax.nn.relu(ssum[0] * inv[0] + root0[0])
    hu = jax.nn.relu(ssum[1] * inv[1] + root0[1])
    tab[0] = jnp.dot(hu, wl_r[...], preferred_element_type=jnp.float32)
    tab[1] = jnp.dot(hi, wl_v[...], preferred_element_type=jnp.float32)
    root[0] = jnp.dot(hi, wr_r[...],
                      preferred_element_type=jnp.float32) + blr[...]
    root[1] = jnp.dot(hu, wr_v[...],
                      preferred_element_type=jnp.float32) + blv[...]


def _tc3_body(ssum, inv, root1, wfu, bfu, wfi, bfi, out_u, out_i):
    hi = jax.nn.relu(ssum[0] * inv[0] + root1[0])
    hu = jax.nn.relu(ssum[1] * inv[1] + root1[1])
    out_u[...] = jnp.dot(hu, wfu[...],
                         preferred_element_type=jnp.float32) + bfu[...]
    out_i[...] = jnp.dot(hi, wfi[...],
                         preferred_element_type=jnp.float32) + bfi[...]


_G = N // RB
_b2 = pl.BlockSpec((2, RB, H), lambda i: (0, i, 0))
_b2i = pl.BlockSpec((2, RB, 1), lambda i: (0, i, 0))
_b1 = pl.BlockSpec((RB, H), lambda i: (i, 0))

_tc1 = pl.pallas_call(
    _tc1_body,
    grid=(_G,),
    in_specs=[
        pl.BlockSpec((RB, DIN), lambda i: (i, 0)),
        pl.BlockSpec((RB, DIN), lambda i: (i, 0)),
        _full((DIN, H)), _full((1, H)), _full((DIN, H)), _full((1, H)),
        _full((H, H)), _full((H, H)), _full((1, H)),
        _full((H, H)), _full((H, H)), _full((1, H)),
    ],
    out_specs=[_b2, _b2],
    out_shape=[jax.ShapeDtypeStruct((2, N, H), jnp.float32)] * 2,
)

_tc2 = pl.pallas_call(
    _tc2_body,
    grid=(_G,),
    in_specs=[
        _b2, _b2i, _b2,
        _full((H, H)), _full((H, H)), _full((1, H)),
        _full((H, H)), _full((H, H)), _full((1, H)),
    ],
    out_specs=[_b2, _b2],
    out_shape=[jax.ShapeDtypeStruct((2, N, H), jnp.float32)] * 2,
)

_tc3 = pl.pallas_call(
    _tc3_body,
    grid=(_G,),
    in_specs=[
        _b2, _b2i, _b2,
        _full((H, H)), _full((1, H)), _full((H, H)), _full((1, H)),
    ],
    out_specs=[_b1, _b1],
    out_shape=[jax.ShapeDtypeStruct((N, H), jnp.float32)] * 2,
)


def _pad_edges(src, dst, src_off):
    pad = EPAD - E
    src = jnp.concatenate(
        [src.astype(jnp.int32) + src_off, jnp.zeros((pad,), jnp.int32)])
    dst = jnp.concatenate(
        [dst.astype(jnp.int32), jnp.full((pad,), DPAD, jnp.int32)])
    return src.reshape(NS, NCHUNK, CH), dst.reshape(NS, NCHUNK, CH)


@jax.jit
def kernel(x_user, x_item, edge_index_rates, edge_index_rev,
           Wp_user, bp_user, Wp_item, bp_item,
           Wl0_rates, bl0_rates, Wr0_rates, Wl0_rev, bl0_rev, Wr0_rev,
           Wl1_rates, bl1_rates, Wr1_rates, Wl1_rev, bl1_rev, Wr1_rev,
           Wf_user, bf_user, Wf_item, bf_item):
    src_r, dst_r = _pad_edges(edge_index_rates[0], edge_index_rates[1], 0)
    src_v, dst_v = _pad_edges(edge_index_rev[0], edge_index_rev[1], N)
    idx_src = jnp.concatenate([src_r[None], src_v[None]]).reshape(
        NC * NS, NCHUNK, CH)
    idx_dst = jnp.concatenate([dst_r[None], dst_v[None]]).reshape(
        NC * NS, NCHUNK, CH)

    b2 = lambda b: b.reshape(1, H)
    tab0, root0 = _tc1(x_user, x_item, Wp_user, b2(bp_user), Wp_item,
                       b2(bp_item), Wl0_rates, Wr0_rates, b2(bl0_rates),
                       Wl0_rev, Wr0_rev, b2(bl0_rev))

    s0, inv_raw = _make_segsum(True)(tab0.reshape(NC * N, H), idx_src,
                                     idx_dst)
    inv = inv_raw.reshape(NC, NACC, L)[:, :, :1]

    tab1, root1 = _tc2(s0, inv, root0, Wl1_rates, Wr1_rates, b2(bl1_rates),
                       Wl1_rev, Wr1_rev, b2(bl1_rev))

    (s1,) = _make_segsum(False)(tab1.reshape(NC * N, H), idx_src, idx_dst)

    out_user, out_item = _tc3(s1, inv, root1, Wf_user, b2(bf_user),
                              Wf_item, b2(bf_item))
    return (out_user, out_item)
